# baseline (speedup 1.0000x reference)
import functools

import jax
import jax.numpy as jnp
from jax import lax
from jax.experimental import pallas as pl
from jax.experimental.pallas import tpu as pltpu

_S, _D, _H, _E, _TOPK, _F = 2048, 1024, 64, 16, 2, 64
_CH = 256
_NC = _S // _CH
_GP_ROWS = _D + _H + _S // 128


def _moe_kernel(x_hbm, gp_ref, wexp_hbm, out_ref,
                x_vmem, xb16, wsel, sem_x, sem_w):
    for c in range(_NC):
        pltpu.make_async_copy(
            x_hbm.at[pl.ds(c * _CH, _CH), :],
            x_vmem.at[pl.ds(c * _CH, _CH), :],
            sem_x.at[c],
        ).start()

    v = jnp.zeros((1, _D), jnp.float32)
    for c in range(_NC):
        pltpu.make_async_copy(
            x_hbm.at[pl.ds(c * _CH, _CH), :],
            x_vmem.at[pl.ds(c * _CH, _CH), :],
            sem_x.at[c],
        ).wait()
        xs = x_vmem[pl.ds(c * _CH, _CH), :]
        xb16[pl.ds(c * _CH, _CH), :] = xs.astype(jnp.bfloat16)
        wrow = gp_ref[_D + _H + 2 * c : _D + _H + 2 * c + 2, :].reshape(1, _CH)
        v = v + jnp.dot(wrow, xs, preferred_element_type=jnp.float32)

    g = jnp.dot(
        jnp.dot(v, gp_ref[0:_D, 0:_H], preferred_element_type=jnp.float32),
        gp_ref[_D : _D + _H, 0:_E],
        preferred_element_type=jnp.float32,
    )

    gi = lax.broadcasted_iota(jnp.int32, (1, _E), 1)
    m1 = jnp.max(g)
    i1 = jnp.min(jnp.where(g == m1, gi, _E))
    g2 = jnp.where(gi == i1, -jnp.inf, g)
    m2 = jnp.max(g2)
    i2 = jnp.min(jnp.where(g2 == m2, gi, _E))
    e21 = jnp.exp(m2 - m1)
    w0 = 1.0 / (1.0 + e21)
    w1 = e21 / (1.0 + e21)

    cp_a = pltpu.make_async_copy(wexp_hbm.at[i1], wsel.at[0], sem_w.at[0])
    cp_b = pltpu.make_async_copy(wexp_hbm.at[i2], wsel.at[1], sem_w.at[1])
    cp_a.start()
    cp_b.start()
    cp_a.wait()
    cp_b.wait()
    wab16 = jnp.concatenate([wsel[0], wsel[1]], axis=1).astype(jnp.bfloat16)

    ri = lax.broadcasted_iota(jnp.int32, (2 * _F, 2 * _F), 0) // _F
    ci = lax.broadcasted_iota(jnp.int32, (2 * _F, 2 * _F), 1) // _F
    bd = (ri == ci).astype(jnp.float32)

    inv_sqrt2 = 0.7071067811865476
    for c in range(_NC):
        xs16 = xb16[pl.ds(c * _CH, _CH), :]
        z = jnp.dot(xs16, wab16, preferred_element_type=jnp.float32)
        s = jnp.dot(z * z, bd, preferred_element_type=jnp.float32)
        n = jnp.maximum(jnp.sqrt(s), 1e-12)
        zn = z / n
        gl = 0.5 * zn * (1.0 + lax.erf(zn * inv_sqrt2))
        out_ref[pl.ds(c * _CH, _CH), :] = w0 * gl[:, 0:_F] + w1 * gl[:, _F : 2 * _F]


@functools.partial(jax.jit, static_argnames=())
def kernel(x, W_gate_in, W_gate_lin, W_gate_out, W_experts):
    gp = jnp.zeros((_GP_ROWS, 128), jnp.float32)
    gp = gp.at[0:_D, 0:_H].set(W_gate_in)
    gp = gp.at[_D : _D + _H, 0:_E].set(W_gate_lin)
    gp = gp.at[_D + _H : _GP_ROWS, :].set(W_gate_out.reshape(_S // 128, 128))
    return pl.pallas_call(
        _moe_kernel,
        out_shape=jax.ShapeDtypeStruct((_S, _F), jnp.float32),
        in_specs=[
            pl.BlockSpec(memory_space=pltpu.MemorySpace.HBM),
            pl.BlockSpec(memory_space=pltpu.MemorySpace.VMEM),
            pl.BlockSpec(memory_space=pltpu.MemorySpace.HBM),
        ],
        out_specs=pl.BlockSpec(memory_space=pltpu.MemorySpace.VMEM),
        scratch_shapes=[
            pltpu.VMEM((_S, _D), jnp.float32),
            pltpu.VMEM((_S, _D), jnp.bfloat16),
            pltpu.VMEM((_TOPK, _D, _F), jnp.float32),
            pltpu.SemaphoreType.DMA((_NC,)),
            pltpu.SemaphoreType.DMA((_TOPK,)),
        ],
        compiler_params=pltpu.CompilerParams(
            vmem_limit_bytes=100 * 1024 * 1024,
        ),
    )(x, gp, W_experts)


# speedup vs baseline: 1.0857x; 1.0857x over previous
import functools

import jax
import jax.numpy as jnp
from jax import lax
from jax.experimental import pallas as pl
from jax.experimental.pallas import tpu as pltpu

_S, _D, _H, _E, _TOPK, _F = 2048, 1024, 64, 16, 2, 64
_CH = 256
_NC = _S // _CH
_R_WGL = _D
_R_WGO = _D + _H
_R_END = _R_WGO + _S // 128


def _moe_kernel(x_ref, wexp_ref, gp_ref, out_ref):
    v = jnp.zeros((1, _D), jnp.float32)
    for c in range(_NC):
        xs = x_ref[pl.ds(c * _CH, _CH), :]
        wrow = gp_ref[_R_WGO + 2 * c : _R_WGO + 2 * c + 2, :].reshape(1, _CH)
        v = v + jnp.dot(wrow, xs, preferred_element_type=jnp.float32)

    g = jnp.dot(
        jnp.dot(v, gp_ref[0:_D, 0:_H], preferred_element_type=jnp.float32),
        gp_ref[_R_WGL : _R_WGL + _H, 0:_E],
        preferred_element_type=jnp.float32,
    )

    gi = lax.broadcasted_iota(jnp.int32, (1, _E), 1)
    m1 = jnp.max(g)
    i1 = jnp.min(jnp.where(g == m1, gi, _E))
    g2 = jnp.where(gi == i1, -jnp.inf, g)
    m2 = jnp.max(g2)
    i2 = jnp.min(jnp.where(g2 == m2, gi, _E))
    e21 = jnp.exp(m2 - m1)
    w0 = 1.0 / (1.0 + e21)
    w1 = e21 / (1.0 + e21)

    wab16 = jnp.concatenate([wexp_ref[i1], wexp_ref[i2]], axis=1).astype(jnp.bfloat16)

    ri = lax.broadcasted_iota(jnp.int32, (2 * _F, 2 * _F), 0) // _F
    ci = lax.broadcasted_iota(jnp.int32, (2 * _F, 2 * _F), 1) // _F
    bd = (ri == ci).astype(jnp.float32)

    inv_sqrt2 = 0.7071067811865476
    for c in range(_NC):
        xs16 = x_ref[pl.ds(c * _CH, _CH), :].astype(jnp.bfloat16)
        z = jnp.dot(xs16, wab16, preferred_element_type=jnp.float32)
        s = jnp.dot(z * z, bd, preferred_element_type=jnp.float32)
        n = jnp.maximum(jnp.sqrt(s), 1e-12)
        zn = z / n
        gl = 0.5 * zn * (1.0 + lax.erf(zn * inv_sqrt2))
        out_ref[pl.ds(c * _CH, _CH), :] = w0 * gl[:, 0:_F] + w1 * gl[:, _F : 2 * _F]


@functools.partial(jax.jit, static_argnames=())
def kernel(x, W_gate_in, W_gate_lin, W_gate_out, W_experts):
    gp = jnp.concatenate(
        [
            jnp.pad(W_gate_in, ((0, 0), (0, 128 - _H))),
            jnp.pad(W_gate_lin, ((0, 0), (0, 128 - _E))),
            W_gate_out.reshape(_S // 128, 128),
        ],
        axis=0,
    )
    return pl.pallas_call(
        _moe_kernel,
        out_shape=jax.ShapeDtypeStruct((_S, _F), jnp.float32),
        in_specs=[
            pl.BlockSpec(memory_space=pltpu.MemorySpace.VMEM),
            pl.BlockSpec(memory_space=pltpu.MemorySpace.VMEM),
            pl.BlockSpec(memory_space=pltpu.MemorySpace.VMEM),
        ],
        out_specs=pl.BlockSpec(memory_space=pltpu.MemorySpace.VMEM),
        compiler_params=pltpu.CompilerParams(
            vmem_limit_bytes=100 * 1024 * 1024,
        ),
    )(x, W_experts, gp)
